# Initial kernel scaffold; baseline (speedup 1.0000x reference)
import jax
import jax.numpy as jnp
from jax.experimental import pallas as pl


def kernel(x, table):
    raise NotImplementedError("write your pallas kernel here")


# speedup vs baseline: 1.1001x; 1.1001x over previous
import functools

import jax
import jax.numpy as jnp
from jax import lax
from jax.experimental import pallas as pl
from jax.experimental.pallas import tpu as pltpu
from jax.experimental.pallas import tpu_sc as plsc


def _build_gather(B, D, b_per_w, chunk, n_chunks, NC):
    mesh = plsc.VectorSubcoreMesh(core_axis_name="c", subcore_axis_name="s")

    @functools.partial(
        pl.kernel,
        mesh=mesh,
        out_type=jax.ShapeDtypeStruct((B, D), jnp.float32),
        scratch_types=[
            pltpu.VMEM((chunk,), jnp.int32),
            pltpu.VMEM((chunk, D), jnp.float32),
            pltpu.SemaphoreType.DMA,
        ],
        compiler_params=pltpu.CompilerParams(use_tc_tiling_on_sc=False),
    )
    def gather_kernel(x_hbm, table_hbm, out_hbm, idx_v, rows_v, sem):
        wid = lax.axis_index("s") * NC + lax.axis_index("c")
        base = wid * b_per_w

        def body(i, carry):
            off = base + i * chunk
            pltpu.sync_copy(x_hbm.at[pl.ds(off, chunk)], idx_v)
            pltpu.async_copy(table_hbm.at[idx_v], rows_v, sem).wait()
            pltpu.sync_copy(rows_v, out_hbm.at[pl.ds(off, chunk)])
            return carry

        lax.fori_loop(0, n_chunks, body, 0)

    return gather_kernel


def kernel(x, table):
    batch, hist = x.shape
    V, D = table.shape
    B = batch * hist

    info = plsc.get_sparse_core_info()
    NC, NS = info.num_cores, info.num_subcores
    NW = NC * NS
    b_per_w = B // NW
    chunk = 1280
    n_chunks = b_per_w // chunk

    flat_idx = x.reshape((B,)).astype(jnp.int32)
    gk = _build_gather(B, D, b_per_w, chunk, n_chunks, NC)
    out = gk(flat_idx, table)
    return out.reshape((batch, hist, D))


# speedup vs baseline: 1.1122x; 1.0110x over previous
import functools

import jax
import jax.numpy as jnp
from jax import lax
from jax.experimental import pallas as pl
from jax.experimental.pallas import tpu as pltpu
from jax.experimental.pallas import tpu_sc as plsc


def _build_gather(B, D, b_per_w, chunk, nbuf, n_groups, NC):
    mesh = plsc.VectorSubcoreMesh(core_axis_name="c", subcore_axis_name="s")

    scratch = (
        [pltpu.VMEM((chunk,), jnp.int32) for _ in range(nbuf)]
        + [pltpu.VMEM((chunk, D), jnp.float32) for _ in range(nbuf)]
        + [pltpu.SemaphoreType.DMA for _ in range(3 * nbuf)]
    )

    @functools.partial(
        pl.kernel,
        mesh=mesh,
        out_type=jax.ShapeDtypeStruct((B, D), jnp.float32),
        scratch_types=scratch,
        compiler_params=pltpu.CompilerParams(use_tc_tiling_on_sc=False),
    )
    def gather_kernel(x_hbm, table_hbm, out_hbm, *bufs):
        idx_v = bufs[:nbuf]
        rows_v = bufs[nbuf:2 * nbuf]
        isem = bufs[2 * nbuf:3 * nbuf]
        gsem = bufs[3 * nbuf:4 * nbuf]
        osem = bufs[4 * nbuf:5 * nbuf]

        wid = lax.axis_index("s") * NC + lax.axis_index("c")
        base = wid * b_per_w

        def start_idx(c, b):
            pltpu.async_copy(x_hbm.at[pl.ds(base + c * chunk, chunk)],
                             idx_v[b], isem[b])

        def wait_idx(b):
            pltpu.make_async_copy(x_hbm.at[pl.ds(base, chunk)],
                                  idx_v[b], isem[b]).wait()

        def start_gather(b):
            pltpu.async_copy(table_hbm.at[idx_v[b]], rows_v[b], gsem[b])

        def wait_gather(b):
            pltpu.make_async_copy(table_hbm.at[idx_v[b]], rows_v[b],
                                  gsem[b]).wait()

        def start_out(c, b):
            pltpu.async_copy(rows_v[b],
                             out_hbm.at[pl.ds(base + c * chunk, chunk)],
                             osem[b])

        def wait_out(b):
            pltpu.make_async_copy(rows_v[b],
                                  out_hbm.at[pl.ds(base, chunk)],
                                  osem[b]).wait()

        start_idx(0, 0)

        def body(g, carry):
            c0 = g * nbuf
            for b in range(1, nbuf):
                start_idx(c0 + b, b)
            for b in range(nbuf):
                wait_idx(b)

                @pl.when(g > 0)
                def _():
                    wait_out(b)

                start_gather(b)
            for b in range(nbuf):
                wait_gather(b)
                start_out(c0 + b, b)
                if b == 0:
                    @pl.when(g < n_groups - 1)
                    def _():
                        start_idx(c0 + nbuf, 0)
            return carry

        lax.fori_loop(0, n_groups, body, 0)
        for b in range(nbuf):
            wait_out(b)

    return gather_kernel


def kernel(x, table):
    batch, hist = x.shape
    V, D = table.shape
    B = batch * hist

    info = plsc.get_sparse_core_info()
    NC, NS = info.num_cores, info.num_subcores
    NW = NC * NS
    b_per_w = B // NW
    chunk = 800
    nbuf = 4
    n_groups = b_per_w // (chunk * nbuf)

    flat_idx = x.reshape((B,)).astype(jnp.int32)
    gk = _build_gather(B, D, b_per_w, chunk, nbuf, n_groups, NC)
    out = gk(flat_idx, table)
    return out.reshape((batch, hist, D))


# speedup vs baseline: 1.8070x; 1.6248x over previous
import functools

import jax
import jax.numpy as jnp
from jax import lax
from jax.experimental import pallas as pl
from jax.experimental.pallas import tpu as pltpu
from jax.experimental.pallas import tpu_sc as plsc


def _build_gather(batch, hist, V, D, b_per_w, chunk, nbuf, n_groups, NC):
    mesh = plsc.VectorSubcoreMesh(core_axis_name="c", subcore_axis_name="s")
    B = batch * hist

    scratch = (
        [pltpu.VMEM((chunk,), jnp.int32) for _ in range(nbuf)]
        + [pltpu.VMEM((chunk, D), jnp.float32) for _ in range(nbuf)]
        + [pltpu.SemaphoreType.DMA for _ in range(3 * nbuf)]
    )

    @functools.partial(
        pl.kernel,
        mesh=mesh,
        out_type=jax.ShapeDtypeStruct((batch, hist, D), jnp.float32),
        scratch_types=scratch,
        compiler_params=pltpu.CompilerParams(use_tc_tiling_on_sc=False),
    )
    def gather_kernel(x_hbm, table_hbm, out_hbm3, *bufs):
        idx_v = bufs[:nbuf]
        rows_v = bufs[nbuf:2 * nbuf]
        isem = bufs[2 * nbuf:3 * nbuf]
        gsem = bufs[3 * nbuf:4 * nbuf]
        osem = bufs[4 * nbuf:5 * nbuf]

        wid = lax.axis_index("s") * NC + lax.axis_index("c")
        base = wid * b_per_w
        xrow0 = wid * (b_per_w // hist)
        rows_per_chunk = chunk // hist

        def start_idx(c, b):
            pltpu.async_copy(x_hbm.at[pl.ds(base + c * chunk, chunk)],
                             idx_v[b], isem[b])

        def wait_idx(b):
            pltpu.make_async_copy(x_hbm.at[pl.ds(base, chunk)],
                                  idx_v[b], isem[b]).wait()

        def start_gather(b):
            pltpu.async_copy(table_hbm.at[idx_v[b]], rows_v[b], gsem[b])

        def wait_gather(b):
            pltpu.make_async_copy(table_hbm.at[idx_v[b]], rows_v[b],
                                  gsem[b]).wait()

        def start_out(c, b):
            for r in range(rows_per_chunk):
                pltpu.async_copy(
                    rows_v[b].at[pl.ds(r * hist, hist), :],
                    out_hbm3.at[xrow0 + c * rows_per_chunk + r],
                    osem[b])

        def wait_out(b):
            pltpu.make_async_copy(table_hbm.at[pl.ds(0, chunk), :],
                                  rows_v[b], osem[b]).wait()

        start_idx(0, 0)

        def body(g, carry):
            c0 = g * nbuf
            for b in range(1, nbuf):
                start_idx(c0 + b, b)
            for b in range(nbuf):
                wait_idx(b)

                @pl.when(g > 0)
                def _():
                    wait_out(b)

                start_gather(b)
            for b in range(nbuf):
                wait_gather(b)
                start_out(c0 + b, b)
                if b == 0:
                    @pl.when(g < n_groups - 1)
                    def _():
                        start_idx(c0 + nbuf, 0)
            return carry

        lax.fori_loop(0, n_groups, body, 0)
        for b in range(nbuf):
            wait_out(b)

    return gather_kernel


def kernel(x, table):
    batch, hist = x.shape
    V, D = table.shape
    B = batch * hist

    info = plsc.get_sparse_core_info()
    NC, NS = info.num_cores, info.num_subcores
    NW = NC * NS
    b_per_w = B // NW
    chunk = 800
    nbuf = 4
    n_groups = b_per_w // (chunk * nbuf)

    flat_idx = x.reshape((B,))
    return _build_gather(batch, hist, V, D, b_per_w, chunk, nbuf,
                         n_groups, NC)(flat_idx, table)
